# baseline (speedup 1.0000x reference)
import functools

import jax
import jax.numpy as jnp
from jax import lax
from jax.experimental import pallas as pl
from jax.experimental.pallas import tpu as pltpu
from jax.experimental.pallas import tpu_sc as plsc

_NC, _NS, _NW = 2, 16, 32
_GC = 80
_ZCH = 200


def _encode_body(x_ref, g_ref, b_ref, WT_ref, lb_ref, WiT_ref, WjT_ref,
                 h_ref, hd_ref, hs_ref):
    x = x_ref[...]
    mu = jnp.mean(x, axis=1, keepdims=True)
    var = jnp.mean((x - mu) ** 2, axis=1, keepdims=True)
    xn = (x - mu) * lax.rsqrt(var + 1e-5) * g_ref[...] + b_ref[...]
    h = xn @ WT_ref[...] + lb_ref[...]
    h_ref[...] = h
    hd_ref[...] = h @ WiT_ref[...]
    hs_ref[...] = h @ WjT_ref[...]


def _edge_body(g_ref, rel_ref,
               wd_ref, b1_ref, W2T_ref, b2_ref, xW1T_ref, xb1_ref,
               xW2T_ref, xb2_ref, m_ref, rc_ref):
    rel = rel_ref[...]
    d2 = jnp.sum(rel * rel, axis=1, keepdims=True)
    pre = g_ref[...] + d2 * wd_ref[...] + b1_ref[...]
    m1 = jax.nn.silu(pre)
    m = jax.nn.silu(m1 @ W2T_ref[...] + b2_ref[...])
    t = jax.nn.silu(m @ xW1T_ref[...] + xb1_ref[...])
    coef = t @ xW2T_ref[...] + xb2_ref[...]
    lane = lax.broadcasted_iota(jnp.int32, rel.shape, 1)
    one3 = jnp.where(lane == 3, 1.0, 0.0)
    m_ref[...] = m
    rc_ref[...] = rel * coef + one3


def _update_body(mode, h_ref, ma_ref, rca_ref, p_ref,
                 hAT_ref, hBT_ref, hb1_ref, hW2T_ref, hb2_ref,
                 wA_ref, wB_ref, pb1_ref,
                 h_out, p_out, aux1_out, aux2_out=None):
    m_agg = ma_ref[0] + ma_ref[1]
    rc = rca_ref[0] + rca_ref[1]
    deg = rc[:, 3:4]
    lane = lax.broadcasted_iota(jnp.int32, rc.shape, 1)
    mask3 = jnp.where(lane < 3, 1.0, 0.0)
    p_out[...] = p_ref[...] + (rc / (deg + 1.0)) * mask3
    h = h_ref[...]
    upd = jax.nn.silu(h @ hAT_ref[...] + m_agg @ hBT_ref[...] + hb1_ref[...])
    h_new = h + upd @ hW2T_ref[...] + hb2_ref[...]
    h_out[...] = h_new
    if mode == "proj":
        aux1_out[...] = h_new @ wA_ref[...]
        aux2_out[...] = h_new @ wB_ref[...]
    else:
        aux1_out[...] = jnp.tanh(h_new @ wA_ref[...] + pb1_ref[...]) @ wB_ref[...]


def _pool_body(nb, h_ref, s_ref, batch_ref,
               u_ref, uWT_ref, ub_ref, ug_ref, ulnb_ref, pid_ref, tab_ref,
               nBT_ref, nAT_ref, nb1_ref, nW2T_ref, nb2_ref,
               rW1T_ref, rb1_ref, rW2T_ref, rb2_ref,
               cW1T_ref, cb1_ref, cW2T_ref, cb2_ref,
               nlog_ref, y_ref, cls_ref,
               hacc_ref, mrow_ref, den_ref, uenc_ref, uc_ref):
    i = pl.program_id(0)
    b = batch_ref[...]
    ids = lax.broadcasted_iota(jnp.int32, (b.shape[0], 32), 1)
    oh = b == ids
    s = s_ref[...]

    @pl.when(i == 0)
    def _uenc():
        ue = jax.nn.relu(u_ref[...] @ uWT_ref[...] + ub_ref[...])
        mu = jnp.mean(ue, axis=1, keepdims=True)
        var = jnp.mean((ue - mu) ** 2, axis=1, keepdims=True)
        ue = (ue - mu) * lax.rsqrt(var + 1e-5) * ug_ref[...] + ulnb_ref[...]
        pids = lax.broadcasted_iota(jnp.int32, (pid_ref.shape[0], tab_ref.shape[0]), 1)
        poh = jnp.where(pid_ref[...] == pids, 1.0, 0.0)
        uenc = jnp.concatenate([ue, poh @ tab_ref[...]], axis=1)
        uenc_ref[...] = uenc
        uc_ref[...] = uenc @ nBT_ref[...]

    @pl.when(i < nb)
    def _max_phase():
        sm = jnp.where(oh, s, -1e30)
        bm = jnp.max(sm, axis=0, keepdims=True)

        @pl.when(i == 0)
        def _():
            mrow_ref[...] = bm

        @pl.when(i > 0)
        def _():
            mrow_ref[...] = jnp.maximum(mrow_ref[...], bm)

    @pl.when((i >= nb) & (i < 2 * nb))
    def _sum_phase():
        e2d = jnp.where(oh, jnp.exp(s - mrow_ref[...]), 0.0)
        bs = jnp.sum(e2d, axis=0, keepdims=True)

        @pl.when(i == nb)
        def _():
            den_ref[...] = bs

        @pl.when(i > nb)
        def _():
            den_ref[...] = den_ref[...] + bs

    @pl.when(i >= 2 * nb)
    def _main_phase():
        h = h_ref[...]
        ohf = jnp.where(oh, 1.0, 0.0)
        den = jnp.maximum(den_ref[...], 1e-30)
        e2n = jnp.where(oh, jnp.exp(s - mrow_ref[...]), 0.0) / den
        contrib = lax.dot_general(e2n, h, (((0,), (0,)), ((), ())))

        @pl.when(i == 2 * nb)
        def _():
            hacc_ref[...] = contrib

        @pl.when(i > 2 * nb)
        def _():
            hacc_ref[...] = hacc_ref[...] + contrib

        pre = h @ nAT_ref[...] + ohf @ uc_ref[...] + nb1_ref[...]
        nlog_ref[...] = jax.nn.relu(pre) @ nW2T_ref[...] + nb2_ref[...]

        @pl.when(i == 3 * nb - 1)
        def _heads():
            joint = jnp.concatenate([hacc_ref[...], uenc_ref[...]], axis=1)
            y_ref[...] = jax.nn.silu(joint @ rW1T_ref[...] + rb1_ref[...]) @ rW2T_ref[...] + rb2_ref[...]
            cls_ref[...] = jax.nn.relu(joint @ cW1T_ref[...] + cb1_ref[...]) @ cW2T_ref[...] + cb2_ref[...]


def _sc_gather(Hd, Hs, pos16, src, dst):
    E = src.shape[0]
    Ew = E // _NW
    n = Ew // _GC
    f32 = jnp.float32
    mesh = plsc.VectorSubcoreMesh(core_axis_name="c", subcore_axis_name="s")

    @functools.partial(
        pl.kernel, mesh=mesh,
        compiler_params=pltpu.CompilerParams(use_tc_tiling_on_sc=False),
        out_type=(
            jax.ShapeDtypeStruct((E, 128), f32),
            jax.ShapeDtypeStruct((E, 16), f32),
        ),
        scratch_types=[
            [pltpu.VMEM((_GC,), jnp.int32)] * 2,
            [pltpu.VMEM((_GC,), jnp.int32)] * 2,
            [pltpu.VMEM((_GC, 128), f32)] * 2,
            [pltpu.VMEM((_GC, 128), f32)] * 2,
            [pltpu.VMEM((_GC, 16), f32)] * 2,
            [pltpu.VMEM((_GC, 16), f32)] * 2,
            [pltpu.SemaphoreType.DMA] * 2,
            [pltpu.SemaphoreType.DMA] * 2,
            [pltpu.SemaphoreType.DMA] * 2,
        ],
    )
    def k(hd_hbm, hs_hbm, p_hbm, src_hbm, dst_hbm,
          g_out, rel_out,
          idxd, idxs, bd, bs, bpd, bps, isem, gsem, wsem):
        wid = lax.axis_index("s") * _NC + lax.axis_index("c")
        base = wid * Ew

        def fire_idx(i, b):
            off = base + i * _GC
            pltpu.async_copy(dst_hbm.at[pl.ds(off, _GC)], idxd[b], isem[b])
            pltpu.async_copy(src_hbm.at[pl.ds(off, _GC)], idxs[b], isem[b])

        def wait_idx(b):
            pltpu.make_async_copy(dst_hbm.at[pl.ds(0, _GC)], idxd[b], isem[b]).wait()
            pltpu.make_async_copy(src_hbm.at[pl.ds(0, _GC)], idxs[b], isem[b]).wait()

        def fire_gather(b):
            pltpu.async_copy(hd_hbm.at[idxd[b]], bd[b], gsem[b])
            pltpu.async_copy(hs_hbm.at[idxs[b]], bs[b], gsem[b])
            pltpu.async_copy(p_hbm.at[idxd[b]], bpd[b], gsem[b])
            pltpu.async_copy(p_hbm.at[idxs[b]], bps[b], gsem[b])

        def wait_gather(b):
            pltpu.make_async_copy(hd_hbm.at[pl.ds(0, _GC)], bd[b], gsem[b]).wait()
            pltpu.make_async_copy(hs_hbm.at[pl.ds(0, _GC)], bs[b], gsem[b]).wait()
            pltpu.make_async_copy(p_hbm.at[pl.ds(0, _GC)], bpd[b], gsem[b]).wait()
            pltpu.make_async_copy(p_hbm.at[pl.ds(0, _GC)], bps[b], gsem[b]).wait()

        def fire_wb(i, b):
            off = base + i * _GC
            pltpu.async_copy(bd[b], g_out.at[pl.ds(off, _GC)], wsem[b])
            pltpu.async_copy(bpd[b], rel_out.at[pl.ds(off, _GC)], wsem[b])

        def wait_wb(b):
            pltpu.make_async_copy(bd[b], g_out.at[pl.ds(0, _GC)], wsem[b]).wait()
            pltpu.make_async_copy(bpd[b], rel_out.at[pl.ds(0, _GC)], wsem[b]).wait()

        def compute(b):
            @plsc.parallel_loop(0, _GC, 1, unroll=4)
            def crow(r):
                for cc in range(8):
                    sl = pl.ds(cc * 16, 16)
                    bd[b][r, sl] = bd[b][r, sl] + bs[b][r, sl]
                bpd[b][r, :] = bpd[b][r, :] - bps[b][r, :]

        fire_idx(0, 0)
        fire_idx(1, 1)
        wait_idx(0)
        fire_gather(0)

        def pair(j, carry):
            for sub in range(2):
                i = 2 * j + sub
                b = sub
                nb = 1 - sub

                @pl.when(i < n)
                def _():
                    @pl.when(i + 1 < n)
                    def _():
                        @pl.when(i >= 1)
                        def _():
                            wait_wb(nb)
                        wait_idx(nb)
                        fire_gather(nb)

                    wait_gather(b)
                    compute(b)
                    fire_wb(i, b)

                    @pl.when(i + 2 < n)
                    def _():
                        fire_idx(i + 2, b)
            return carry

        lax.fori_loop(0, (n + 1) // 2, pair, 0)
        wait_wb((n - 2) % 2)
        wait_wb((n - 1) % 2)

    return k(Hd, Hs, pos16, src, dst)


def _sc_scatter2(m0, rc0, dst0, m1, rc1, dst1, N, zero128, zero16):
    E0, E1 = dst0.shape[0], dst1.shape[0]
    nz = N // _ZCH
    f32 = jnp.float32
    mesh = plsc.VectorSubcoreMesh(core_axis_name="c", subcore_axis_name="s")

    @functools.partial(
        pl.kernel, mesh=mesh,
        compiler_params=pltpu.CompilerParams(use_tc_tiling_on_sc=False),
        out_type=(
            jax.ShapeDtypeStruct((2, N, 128), f32),
            jax.ShapeDtypeStruct((2, N, 16), f32),
        ),
        scratch_types=[
            [pltpu.VMEM((_GC,), jnp.int32)] * 2,
            [pltpu.VMEM((_GC, 128), f32)] * 2,
            [pltpu.VMEM((_GC, 16), f32)] * 2,
            pltpu.VMEM_SHARED((N, 128), f32),
            pltpu.VMEM_SHARED((N, 16), f32),
            [pltpu.SemaphoreType.DMA] * 2,
            [pltpu.SemaphoreType.DMA] * 2,
        ],
    )
    def k(m0_hbm, rc0_hbm, dst0_hbm, m1_hbm, rc1_hbm, dst1_hbm,
          z128_hbm, z16_hbm,
          ma_out, rca_out,
          idx, bm, brc, shm, shrc, lsem, ssem):
        c = lax.axis_index("c")
        s = lax.axis_index("s")
        wid = s * _NC + c

        def zbody(j, carry):
            @pl.when(j % _NS == s)
            def _():
                pltpu.sync_copy(z128_hbm, shm.at[pl.ds(j * _ZCH, _ZCH)])
                pltpu.sync_copy(z16_hbm, shrc.at[pl.ds(j * _ZCH, _ZCH)])
            return carry

        lax.fori_loop(0, nz, zbody, 0)
        plsc.subcore_barrier()

        def run_stream(m_hbm, rc_hbm, dst_hbm, Ew, nchunks):
            base = wid * Ew

            def fire_load(i, b):
                off = base + i * _GC
                pltpu.async_copy(dst_hbm.at[pl.ds(off, _GC)], idx[b], lsem[b])
                pltpu.async_copy(m_hbm.at[pl.ds(off, _GC)], bm[b], lsem[b])
                pltpu.async_copy(rc_hbm.at[pl.ds(off, _GC)], brc[b], lsem[b])

            def wait_load(b):
                pltpu.make_async_copy(dst_hbm.at[pl.ds(0, _GC)], idx[b], lsem[b]).wait()
                pltpu.make_async_copy(m_hbm.at[pl.ds(0, _GC)], bm[b], lsem[b]).wait()
                pltpu.make_async_copy(rc_hbm.at[pl.ds(0, _GC)], brc[b], lsem[b]).wait()

            def fire_scatter(b):
                pltpu.async_copy(bm[b], shm.at[idx[b]], ssem[b], add=True)
                pltpu.async_copy(brc[b], shrc.at[idx[b]], ssem[b], add=True)

            def wait_scatter(b):
                pltpu.make_async_copy(m0_hbm.at[pl.ds(0, _GC)], bm[b], ssem[b]).wait()
                pltpu.make_async_copy(rc0_hbm.at[pl.ds(0, _GC)], brc[b], ssem[b]).wait()

            fire_load(0, 0)

            def pair(j, carry):
                for sub in range(2):
                    i = 2 * j + sub
                    b = sub
                    nb = 1 - sub

                    @pl.when(i < nchunks)
                    def _():
                        @pl.when(i + 1 < nchunks)
                        def _():
                            @pl.when(i >= 1)
                            def _():
                                wait_scatter(nb)
                            fire_load(i + 1, nb)

                        wait_load(b)
                        fire_scatter(b)
                return carry

            lax.fori_loop(0, (nchunks + 1) // 2, pair, 0)
            wait_scatter((nchunks - 2) % 2)
            wait_scatter((nchunks - 1) % 2)

        run_stream(m0_hbm, rc0_hbm, dst0_hbm, E0 // _NW, E0 // _NW // _GC)
        run_stream(m1_hbm, rc1_hbm, dst1_hbm, E1 // _NW, E1 // _NW // _GC)
        plsc.subcore_barrier()

        def dbody(j, carry):
            @pl.when(j % _NS == s)
            def _():
                pltpu.sync_copy(shm.at[pl.ds(j * _ZCH, _ZCH)],
                                ma_out.at[c, pl.ds(j * _ZCH, _ZCH)])
                pltpu.sync_copy(shrc.at[pl.ds(j * _ZCH, _ZCH)],
                                rca_out.at[c, pl.ds(j * _ZCH, _ZCH)])
            return carry

        lax.fori_loop(0, nz, dbody, 0)

    return k(m0, rc0, dst0, m1, rc1, dst1, zero128, zero16)


def _row(v):
    return v.reshape(1, -1)


def kernel(x, edge_index, pos, u, batch, pid, params):
    N, in_dim = x.shape
    E = edge_index.shape[1]
    B = u.shape[0]
    f32 = jnp.float32
    src, dst = edge_index[0], edge_index[1]

    BN = 2000
    BE = 2560
    nbN = N // BN

    pos16 = jnp.pad(pos, ((0, 0), (0, 13)))
    batch2d = batch.reshape(N, 1)
    pid2d = pid.reshape(B, 1)
    zero128 = jnp.zeros((_ZCH, 128), f32)
    zero16 = jnp.zeros((_ZCH, 16), f32)

    p = params
    lps = p["layers"]

    def spec(shape, im=None):
        return pl.BlockSpec(shape, im if im is not None else (lambda i: tuple(0 for _ in shape)))

    wspec = lambda shape: spec(shape)

    l0 = lps[0]
    h, Hd, Hs = pl.pallas_call(
        _encode_body,
        grid=(nbN,),
        in_specs=[
            spec((BN, in_dim), lambda i: (i, 0)),
            wspec((1, in_dim)), wspec((1, in_dim)),
            wspec((in_dim, 128)), wspec((1, 128)),
            wspec((128, 128)), wspec((128, 128)),
        ],
        out_specs=[spec((BN, 128), lambda i: (i, 0))] * 3,
        out_shape=[jax.ShapeDtypeStruct((N, 128), f32)] * 3,
    )(x, _row(p["ln_g"]), _row(p["ln_b"]), p["lin_W"].T, _row(p["lin_b"]),
      l0["eW1"][:, :128].T, l0["eW1"][:, 128:256].T)

    qE = E // (5 * _GC * _NW) * (3 * _GC * _NW)
    splits = [(0, qE), (qE, E - qE)]
    src_h = [src[o:o + n] for o, n in splits]
    dst_h = [dst[o:o + n] for o, n in splits]

    for li, lp in enumerate(lps):
        last = li == len(lps) - 1
        Gr = [_sc_gather(Hd, Hs, pos16, src_h[hx], dst_h[hx]) for hx in range(2)]
        mrcs = []
        for hx in range(2):
            Eh = splits[hx][1]
            m_h, rc_h = pl.pallas_call(
                _edge_body,
                grid=(Eh // BE,),
                in_specs=[
                    spec((BE, 128), lambda i: (i, 0)),
                    spec((BE, 16), lambda i: (i, 0)),
                    wspec((1, 128)), wspec((1, 128)),
                    wspec((128, 128)), wspec((1, 128)),
                    wspec((128, 128)), wspec((1, 128)),
                    wspec((128, 1)), wspec((1, 1)),
                ],
                out_specs=[spec((BE, 128), lambda i: (i, 0)),
                           spec((BE, 16), lambda i: (i, 0))],
                out_shape=[jax.ShapeDtypeStruct((Eh, 128), f32),
                           jax.ShapeDtypeStruct((Eh, 16), f32)],
            )(Gr[hx][0], Gr[hx][1],
              _row(lp["eW1"][:, 256]), _row(lp["eb1"]),
              lp["eW2"].T, _row(lp["eb2"]),
              lp["xW1"].T, _row(lp["xb1"]),
              lp["xW2"].T, _row(lp["xb2"]))
            mrcs.append((m_h, rc_h))

        ma, rca = _sc_scatter2(mrcs[0][0], mrcs[0][1], dst_h[0],
                               mrcs[1][0], mrcs[1][1], dst_h[1],
                               N, zero128, zero16)

        if not last:
            nxt = lps[li + 1]
            wA = nxt["eW1"][:, :128].T
            wB = nxt["eW1"][:, 128:256].T
            pb1 = jnp.zeros((1, 128), f32)
            out_shape = [jax.ShapeDtypeStruct((N, 128), f32),
                         jax.ShapeDtypeStruct((N, 16), f32),
                         jax.ShapeDtypeStruct((N, 128), f32),
                         jax.ShapeDtypeStruct((N, 128), f32)]
            out_specs = [spec((BN, 128), lambda i: (i, 0)),
                         spec((BN, 16), lambda i: (i, 0)),
                         spec((BN, 128), lambda i: (i, 0)),
                         spec((BN, 128), lambda i: (i, 0))]
            mode = "proj"
        else:
            wA = p["pW1"].T
            wB = p["pW2"].T
            pb1 = _row(p["pb1"])
            out_shape = [jax.ShapeDtypeStruct((N, 128), f32),
                         jax.ShapeDtypeStruct((N, 16), f32),
                         jax.ShapeDtypeStruct((N, 1), f32)]
            out_specs = [spec((BN, 128), lambda i: (i, 0)),
                         spec((BN, 16), lambda i: (i, 0)),
                         spec((BN, 1), lambda i: (i, 0))]
            mode = "attn"

        outs = pl.pallas_call(
            functools.partial(_update_body, mode),
            grid=(nbN,),
            in_specs=[
                spec((BN, 128), lambda i: (i, 0)),
                spec((2, BN, 128), lambda i: (0, i, 0)),
                spec((2, BN, 16), lambda i: (0, i, 0)),
                spec((BN, 16), lambda i: (i, 0)),
                wspec((128, 128)), wspec((128, 128)), wspec((1, 128)),
                wspec((128, 128)), wspec((1, 128)),
                wspec(wA.shape), wspec(wB.shape), wspec(pb1.shape),
            ],
            out_specs=out_specs,
            out_shape=out_shape,
        )(h, ma, rca, pos16,
          lp["hW1"][:, :128].T, lp["hW1"][:, 128:].T, _row(lp["hb1"]),
          lp["hW2"].T, _row(lp["hb2"]),
          wA, wB, pb1)
        if not last:
            h, pos16, Hd, Hs = outs
        else:
            h, pos16, s_attn = outs

    nlog, y, cls = pl.pallas_call(
        functools.partial(_pool_body, nbN),
        grid=(3 * nbN,),
        in_specs=[
            spec((BN, 128), lambda i: (jnp.maximum(i - 2 * nbN, 0), 0)),
            spec((BN, 1), lambda i: (i % nbN, 0)),
            spec((BN, 1), lambda i: (i % nbN, 0)),
            wspec((B, 16)), wspec((16, 32)), wspec((1, 32)),
            wspec((1, 32)), wspec((1, 32)),
            wspec((B, 1)), wspec((1000, 32)), wspec((64, 128)),
            wspec((128, 128)), wspec((1, 128)),
            wspec((128, 1)), wspec((1, 1)),
            wspec((192, 128)), wspec((1, 128)),
            wspec((128, 1)), wspec((1, 1)),
            wspec((192, 128)), wspec((1, 128)),
            wspec((128, 1)), wspec((1, 1)),
        ],
        out_specs=[spec((BN, 1), lambda i: (jnp.maximum(i - 2 * nbN, 0), 0)),
                   spec((B, 1)), spec((B, 1))],
        out_shape=[jax.ShapeDtypeStruct((N, 1), f32),
                   jax.ShapeDtypeStruct((B, 1), f32),
                   jax.ShapeDtypeStruct((B, 1), f32)],
        scratch_shapes=[pltpu.VMEM((B, 128), f32),
                        pltpu.VMEM((1, 32), f32),
                        pltpu.VMEM((1, 32), f32),
                        pltpu.VMEM((B, 64), f32),
                        pltpu.VMEM((B, 128), f32)],
    )(h, s_attn, batch2d,
      u, p["uW"].T, _row(p["ub"]), _row(p["u_ln_g"]), _row(p["u_ln_b"]),
      pid2d, p["pid_table"], p["nW1"][:, 128:].T,
      p["nW1"][:, :128].T, _row(p["nb1"]), p["nW2"].T, _row(p["nb2"]),
      p["rW1"].T, _row(p["rb1"]), p["rW2"].T, _row(p["rb2"]),
      p["cW1"].T, _row(p["cb1"]), p["cW2"].T, _row(p["cb2"]))

    return (y[:, 0], cls[:, 0], nlog[:, 0], h, pos16[:, :3])


# speedup vs baseline: 1.0328x; 1.0328x over previous
import functools

import jax
import jax.numpy as jnp
from jax import lax
from jax.experimental import pallas as pl
from jax.experimental.pallas import tpu as pltpu
from jax.experimental.pallas import tpu_sc as plsc

_NC, _NS, _NW = 2, 16, 32
_GC = 80
_ZCH = 200


def _encode_body(x_ref, g_ref, b_ref, WT_ref, lb_ref, WiT_ref, WjT_ref,
                 h_ref, hd_ref, hs_ref):
    x = x_ref[...]
    mu = jnp.mean(x, axis=1, keepdims=True)
    var = jnp.mean((x - mu) ** 2, axis=1, keepdims=True)
    xn = (x - mu) * lax.rsqrt(var + 1e-5) * g_ref[...] + b_ref[...]
    h = xn @ WT_ref[...] + lb_ref[...]
    h_ref[...] = h
    hd_ref[...] = h @ WiT_ref[...]
    hs_ref[...] = h @ WjT_ref[...]


def _edge_body(g_ref, rel_ref,
               wd_ref, b1_ref, W2T_ref, b2_ref, xW1T_ref, xb1_ref,
               xW2T_ref, xb2_ref, m_ref, rc_ref):
    rel = rel_ref[...]
    d2 = jnp.sum(rel * rel, axis=1, keepdims=True)
    pre = g_ref[...] + d2 * wd_ref[...] + b1_ref[...]
    m1 = jax.nn.silu(pre)
    m = jax.nn.silu(m1 @ W2T_ref[...] + b2_ref[...])
    t = jax.nn.silu(m @ xW1T_ref[...] + xb1_ref[...])
    coef = t @ xW2T_ref[...] + xb2_ref[...]
    lane = lax.broadcasted_iota(jnp.int32, rel.shape, 1)
    one3 = jnp.where(lane == 3, 1.0, 0.0)
    m_ref[...] = m
    rc_ref[...] = rel * coef + one3


def _update_body(mode, h_ref, ma0_ref, ma1_ref, rca0_ref, rca1_ref, p_ref,
                 hAT_ref, hBT_ref, hb1_ref, hW2T_ref, hb2_ref,
                 wA_ref, wB_ref, pb1_ref,
                 h_out, p_out, aux1_out, aux2_out=None):
    m_agg = ma0_ref[0] + ma0_ref[1] + ma1_ref[0] + ma1_ref[1]
    rc = rca0_ref[0] + rca0_ref[1] + rca1_ref[0] + rca1_ref[1]
    deg = rc[:, 3:4]
    lane = lax.broadcasted_iota(jnp.int32, rc.shape, 1)
    mask3 = jnp.where(lane < 3, 1.0, 0.0)
    p_out[...] = p_ref[...] + (rc / (deg + 1.0)) * mask3
    h = h_ref[...]
    upd = jax.nn.silu(h @ hAT_ref[...] + m_agg @ hBT_ref[...] + hb1_ref[...])
    h_new = h + upd @ hW2T_ref[...] + hb2_ref[...]
    h_out[...] = h_new
    if mode == "proj":
        aux1_out[...] = h_new @ wA_ref[...]
        aux2_out[...] = h_new @ wB_ref[...]
    else:
        aux1_out[...] = jnp.tanh(h_new @ wA_ref[...] + pb1_ref[...]) @ wB_ref[...]


def _pool_body(nb, h_ref, s_ref, batch_ref,
               u_ref, uWT_ref, ub_ref, ug_ref, ulnb_ref, pid_ref, tab_ref,
               nBT_ref, nAT_ref, nb1_ref, nW2T_ref, nb2_ref,
               rW1T_ref, rb1_ref, rW2T_ref, rb2_ref,
               cW1T_ref, cb1_ref, cW2T_ref, cb2_ref,
               nlog_ref, y_ref, cls_ref,
               hacc_ref, mrow_ref, den_ref, uenc_ref, uc_ref):
    i = pl.program_id(0)
    b = batch_ref[...]
    ids = lax.broadcasted_iota(jnp.int32, (b.shape[0], 32), 1)
    oh = b == ids
    s = s_ref[...]

    @pl.when(i == 0)
    def _uenc():
        ue = jax.nn.relu(u_ref[...] @ uWT_ref[...] + ub_ref[...])
        mu = jnp.mean(ue, axis=1, keepdims=True)
        var = jnp.mean((ue - mu) ** 2, axis=1, keepdims=True)
        ue = (ue - mu) * lax.rsqrt(var + 1e-5) * ug_ref[...] + ulnb_ref[...]
        pids = lax.broadcasted_iota(jnp.int32, (pid_ref.shape[0], tab_ref.shape[0]), 1)
        poh = jnp.where(pid_ref[...] == pids, 1.0, 0.0)
        uenc = jnp.concatenate([ue, poh @ tab_ref[...]], axis=1)
        uenc_ref[...] = uenc
        uc_ref[...] = uenc @ nBT_ref[...]

    @pl.when(i < nb)
    def _max_phase():
        sm = jnp.where(oh, s, -1e30)
        bm = jnp.max(sm, axis=0, keepdims=True)

        @pl.when(i == 0)
        def _():
            mrow_ref[...] = bm

        @pl.when(i > 0)
        def _():
            mrow_ref[...] = jnp.maximum(mrow_ref[...], bm)

    @pl.when((i >= nb) & (i < 2 * nb))
    def _sum_phase():
        e2d = jnp.where(oh, jnp.exp(s - mrow_ref[...]), 0.0)
        bs = jnp.sum(e2d, axis=0, keepdims=True)

        @pl.when(i == nb)
        def _():
            den_ref[...] = bs

        @pl.when(i > nb)
        def _():
            den_ref[...] = den_ref[...] + bs

    @pl.when(i >= 2 * nb)
    def _main_phase():
        h = h_ref[...]
        ohf = jnp.where(oh, 1.0, 0.0)
        den = jnp.maximum(den_ref[...], 1e-30)
        e2n = jnp.where(oh, jnp.exp(s - mrow_ref[...]), 0.0) / den
        contrib = lax.dot_general(e2n, h, (((0,), (0,)), ((), ())))

        @pl.when(i == 2 * nb)
        def _():
            hacc_ref[...] = contrib

        @pl.when(i > 2 * nb)
        def _():
            hacc_ref[...] = hacc_ref[...] + contrib

        pre = h @ nAT_ref[...] + ohf @ uc_ref[...] + nb1_ref[...]
        nlog_ref[...] = jax.nn.relu(pre) @ nW2T_ref[...] + nb2_ref[...]

        @pl.when(i == 3 * nb - 1)
        def _heads():
            joint = jnp.concatenate([hacc_ref[...], uenc_ref[...]], axis=1)
            y_ref[...] = jax.nn.silu(joint @ rW1T_ref[...] + rb1_ref[...]) @ rW2T_ref[...] + rb2_ref[...]
            cls_ref[...] = jax.nn.relu(joint @ cW1T_ref[...] + cb1_ref[...]) @ cW2T_ref[...] + cb2_ref[...]


def _sc_gather(Hd, Hs, pos16, src, dst):
    E = src.shape[0]
    Ew = E // _NW
    n = Ew // _GC
    f32 = jnp.float32
    mesh = plsc.VectorSubcoreMesh(core_axis_name="c", subcore_axis_name="s")

    @functools.partial(
        pl.kernel, mesh=mesh,
        compiler_params=pltpu.CompilerParams(use_tc_tiling_on_sc=False),
        out_type=(
            jax.ShapeDtypeStruct((E, 128), f32),
            jax.ShapeDtypeStruct((E, 16), f32),
        ),
        scratch_types=[
            [pltpu.VMEM((_GC,), jnp.int32)] * 2,
            [pltpu.VMEM((_GC,), jnp.int32)] * 2,
            [pltpu.VMEM((_GC, 128), f32)] * 2,
            [pltpu.VMEM((_GC, 128), f32)] * 2,
            [pltpu.VMEM((_GC, 16), f32)] * 2,
            [pltpu.VMEM((_GC, 16), f32)] * 2,
            [pltpu.SemaphoreType.DMA] * 2,
            [pltpu.SemaphoreType.DMA] * 2,
            [pltpu.SemaphoreType.DMA] * 2,
        ],
    )
    def k(hd_hbm, hs_hbm, p_hbm, src_hbm, dst_hbm,
          g_out, rel_out,
          idxd, idxs, bd, bs, bpd, bps, isem, gsem, wsem):
        wid = lax.axis_index("s") * _NC + lax.axis_index("c")
        base = wid * Ew

        def fire_idx(i, b):
            off = base + i * _GC
            pltpu.async_copy(dst_hbm.at[pl.ds(off, _GC)], idxd[b], isem[b])
            pltpu.async_copy(src_hbm.at[pl.ds(off, _GC)], idxs[b], isem[b])

        def wait_idx(b):
            pltpu.make_async_copy(dst_hbm.at[pl.ds(0, _GC)], idxd[b], isem[b]).wait()
            pltpu.make_async_copy(src_hbm.at[pl.ds(0, _GC)], idxs[b], isem[b]).wait()

        def fire_gather(b):
            pltpu.async_copy(hd_hbm.at[idxd[b]], bd[b], gsem[b])
            pltpu.async_copy(hs_hbm.at[idxs[b]], bs[b], gsem[b])
            pltpu.async_copy(p_hbm.at[idxd[b]], bpd[b], gsem[b])
            pltpu.async_copy(p_hbm.at[idxs[b]], bps[b], gsem[b])

        def wait_gather(b):
            pltpu.make_async_copy(hd_hbm.at[pl.ds(0, _GC)], bd[b], gsem[b]).wait()
            pltpu.make_async_copy(hs_hbm.at[pl.ds(0, _GC)], bs[b], gsem[b]).wait()
            pltpu.make_async_copy(p_hbm.at[pl.ds(0, _GC)], bpd[b], gsem[b]).wait()
            pltpu.make_async_copy(p_hbm.at[pl.ds(0, _GC)], bps[b], gsem[b]).wait()

        def fire_wb(i, b):
            off = base + i * _GC
            pltpu.async_copy(bd[b], g_out.at[pl.ds(off, _GC)], wsem[b])
            pltpu.async_copy(bpd[b], rel_out.at[pl.ds(off, _GC)], wsem[b])

        def wait_wb(b):
            pltpu.make_async_copy(bd[b], g_out.at[pl.ds(0, _GC)], wsem[b]).wait()
            pltpu.make_async_copy(bpd[b], rel_out.at[pl.ds(0, _GC)], wsem[b]).wait()

        def compute(b):
            @plsc.parallel_loop(0, _GC, 1, unroll=4)
            def crow(r):
                for cc in range(8):
                    sl = pl.ds(cc * 16, 16)
                    bd[b][r, sl] = bd[b][r, sl] + bs[b][r, sl]
                bpd[b][r, :] = bpd[b][r, :] - bps[b][r, :]

        fire_idx(0, 0)
        fire_idx(1, 1)
        wait_idx(0)
        fire_gather(0)

        def pair(j, carry):
            for sub in range(2):
                i = 2 * j + sub
                b = sub
                nb = 1 - sub

                @pl.when(i < n)
                def _():
                    @pl.when(i + 1 < n)
                    def _():
                        @pl.when(i >= 1)
                        def _():
                            wait_wb(nb)
                        wait_idx(nb)
                        fire_gather(nb)

                    wait_gather(b)
                    compute(b)
                    fire_wb(i, b)

                    @pl.when(i + 2 < n)
                    def _():
                        fire_idx(i + 2, b)
            return carry

        lax.fori_loop(0, (n + 1) // 2, pair, 0)
        wait_wb((n - 2) % 2)
        wait_wb((n - 1) % 2)

    return k(Hd, Hs, pos16, src, dst)


def _sc_scatter(m0, rc0, dst0, N, zero128, zero16):
    E0 = dst0.shape[0]
    nz = N // _ZCH
    f32 = jnp.float32
    mesh = plsc.VectorSubcoreMesh(core_axis_name="c", subcore_axis_name="s")

    @functools.partial(
        pl.kernel, mesh=mesh,
        compiler_params=pltpu.CompilerParams(use_tc_tiling_on_sc=False),
        out_type=(
            jax.ShapeDtypeStruct((2, N, 128), f32),
            jax.ShapeDtypeStruct((2, N, 16), f32),
        ),
        scratch_types=[
            [pltpu.VMEM((_GC,), jnp.int32)] * 2,
            [pltpu.VMEM((_GC, 128), f32)] * 2,
            [pltpu.VMEM((_GC, 16), f32)] * 2,
            pltpu.VMEM_SHARED((N, 128), f32),
            pltpu.VMEM_SHARED((N, 16), f32),
            [pltpu.SemaphoreType.DMA] * 2,
            [pltpu.SemaphoreType.DMA] * 2,
        ],
    )
    def k(m0_hbm, rc0_hbm, dst0_hbm,
          z128_hbm, z16_hbm,
          ma_out, rca_out,
          idx, bm, brc, shm, shrc, lsem, ssem):
        c = lax.axis_index("c")
        s = lax.axis_index("s")
        wid = s * _NC + c

        def zbody(j, carry):
            @pl.when(j % _NS == s)
            def _():
                pltpu.sync_copy(z128_hbm, shm.at[pl.ds(j * _ZCH, _ZCH)])
                pltpu.sync_copy(z16_hbm, shrc.at[pl.ds(j * _ZCH, _ZCH)])
            return carry

        lax.fori_loop(0, nz, zbody, 0)
        plsc.subcore_barrier()

        def run_stream(m_hbm, rc_hbm, dst_hbm, Ew, nchunks):
            base = wid * Ew

            def fire_load(i, b):
                off = base + i * _GC
                pltpu.async_copy(dst_hbm.at[pl.ds(off, _GC)], idx[b], lsem[b])
                pltpu.async_copy(m_hbm.at[pl.ds(off, _GC)], bm[b], lsem[b])
                pltpu.async_copy(rc_hbm.at[pl.ds(off, _GC)], brc[b], lsem[b])

            def wait_load(b):
                pltpu.make_async_copy(dst_hbm.at[pl.ds(0, _GC)], idx[b], lsem[b]).wait()
                pltpu.make_async_copy(m_hbm.at[pl.ds(0, _GC)], bm[b], lsem[b]).wait()
                pltpu.make_async_copy(rc_hbm.at[pl.ds(0, _GC)], brc[b], lsem[b]).wait()

            def fire_scatter(b):
                pltpu.async_copy(bm[b], shm.at[idx[b]], ssem[b], add=True)
                pltpu.async_copy(brc[b], shrc.at[idx[b]], ssem[b], add=True)

            def wait_scatter(b):
                pltpu.make_async_copy(m0_hbm.at[pl.ds(0, _GC)], bm[b], ssem[b]).wait()
                pltpu.make_async_copy(rc0_hbm.at[pl.ds(0, _GC)], brc[b], ssem[b]).wait()

            fire_load(0, 0)

            def pair(j, carry):
                for sub in range(2):
                    i = 2 * j + sub
                    b = sub
                    nb = 1 - sub

                    @pl.when(i < nchunks)
                    def _():
                        @pl.when(i + 1 < nchunks)
                        def _():
                            @pl.when(i >= 1)
                            def _():
                                wait_scatter(nb)
                            fire_load(i + 1, nb)

                        wait_load(b)
                        fire_scatter(b)
                return carry

            lax.fori_loop(0, (nchunks + 1) // 2, pair, 0)
            wait_scatter((nchunks - 2) % 2)
            wait_scatter((nchunks - 1) % 2)

        run_stream(m0_hbm, rc0_hbm, dst0_hbm, E0 // _NW, E0 // _NW // _GC)
        plsc.subcore_barrier()

        def dbody(j, carry):
            @pl.when(j % _NS == s)
            def _():
                pltpu.sync_copy(shm.at[pl.ds(j * _ZCH, _ZCH)],
                                ma_out.at[c, pl.ds(j * _ZCH, _ZCH)])
                pltpu.sync_copy(shrc.at[pl.ds(j * _ZCH, _ZCH)],
                                rca_out.at[c, pl.ds(j * _ZCH, _ZCH)])
            return carry

        lax.fori_loop(0, nz, dbody, 0)

    return k(m0, rc0, dst0, zero128, zero16)


def _row(v):
    return v.reshape(1, -1)


def kernel(x, edge_index, pos, u, batch, pid, params):
    N, in_dim = x.shape
    E = edge_index.shape[1]
    B = u.shape[0]
    f32 = jnp.float32
    src, dst = edge_index[0], edge_index[1]

    BN = 2000
    BE = 2560
    nbN = N // BN

    pos16 = jnp.pad(pos, ((0, 0), (0, 13)))
    batch2d = batch.reshape(N, 1)
    pid2d = pid.reshape(B, 1)
    zero128 = jnp.zeros((_ZCH, 128), f32)
    zero16 = jnp.zeros((_ZCH, 16), f32)

    p = params
    lps = p["layers"]

    def spec(shape, im=None):
        return pl.BlockSpec(shape, im if im is not None else (lambda i: tuple(0 for _ in shape)))

    wspec = lambda shape: spec(shape)

    l0 = lps[0]
    h, Hd, Hs = pl.pallas_call(
        _encode_body,
        grid=(nbN,),
        in_specs=[
            spec((BN, in_dim), lambda i: (i, 0)),
            wspec((1, in_dim)), wspec((1, in_dim)),
            wspec((in_dim, 128)), wspec((1, 128)),
            wspec((128, 128)), wspec((128, 128)),
        ],
        out_specs=[spec((BN, 128), lambda i: (i, 0))] * 3,
        out_shape=[jax.ShapeDtypeStruct((N, 128), f32)] * 3,
    )(x, _row(p["ln_g"]), _row(p["ln_b"]), p["lin_W"].T, _row(p["lin_b"]),
      l0["eW1"][:, :128].T, l0["eW1"][:, 128:256].T)

    qE = E // (5 * _GC * _NW) * (3 * _GC * _NW)
    splits = [(0, qE), (qE, E - qE)]
    src_h = [src[o:o + n] for o, n in splits]
    dst_h = [dst[o:o + n] for o, n in splits]

    for li, lp in enumerate(lps):
        last = li == len(lps) - 1
        Gr = [_sc_gather(Hd, Hs, pos16, src_h[hx], dst_h[hx]) for hx in range(2)]
        mrcs = []
        for hx in range(2):
            Eh = splits[hx][1]
            m_h, rc_h = pl.pallas_call(
                _edge_body,
                grid=(Eh // BE,),
                in_specs=[
                    spec((BE, 128), lambda i: (i, 0)),
                    spec((BE, 16), lambda i: (i, 0)),
                    wspec((1, 128)), wspec((1, 128)),
                    wspec((128, 128)), wspec((1, 128)),
                    wspec((128, 128)), wspec((1, 128)),
                    wspec((128, 1)), wspec((1, 1)),
                ],
                out_specs=[spec((BE, 128), lambda i: (i, 0)),
                           spec((BE, 16), lambda i: (i, 0))],
                out_shape=[jax.ShapeDtypeStruct((Eh, 128), f32),
                           jax.ShapeDtypeStruct((Eh, 16), f32)],
            )(Gr[hx][0], Gr[hx][1],
              _row(lp["eW1"][:, 256]), _row(lp["eb1"]),
              lp["eW2"].T, _row(lp["eb2"]),
              lp["xW1"].T, _row(lp["xb1"]),
              lp["xW2"].T, _row(lp["xb2"]))
            mrcs.append((m_h, rc_h))

        parts = [_sc_scatter(mrcs[hx][0], mrcs[hx][1], dst_h[hx],
                             N, zero128, zero16) for hx in range(2)]

        if not last:
            nxt = lps[li + 1]
            wA = nxt["eW1"][:, :128].T
            wB = nxt["eW1"][:, 128:256].T
            pb1 = jnp.zeros((1, 128), f32)
            out_shape = [jax.ShapeDtypeStruct((N, 128), f32),
                         jax.ShapeDtypeStruct((N, 16), f32),
                         jax.ShapeDtypeStruct((N, 128), f32),
                         jax.ShapeDtypeStruct((N, 128), f32)]
            out_specs = [spec((BN, 128), lambda i: (i, 0)),
                         spec((BN, 16), lambda i: (i, 0)),
                         spec((BN, 128), lambda i: (i, 0)),
                         spec((BN, 128), lambda i: (i, 0))]
            mode = "proj"
        else:
            wA = p["pW1"].T
            wB = p["pW2"].T
            pb1 = _row(p["pb1"])
            out_shape = [jax.ShapeDtypeStruct((N, 128), f32),
                         jax.ShapeDtypeStruct((N, 16), f32),
                         jax.ShapeDtypeStruct((N, 1), f32)]
            out_specs = [spec((BN, 128), lambda i: (i, 0)),
                         spec((BN, 16), lambda i: (i, 0)),
                         spec((BN, 1), lambda i: (i, 0))]
            mode = "attn"

        outs = pl.pallas_call(
            functools.partial(_update_body, mode),
            grid=(nbN,),
            in_specs=[
                spec((BN, 128), lambda i: (i, 0)),
                spec((2, BN, 128), lambda i: (0, i, 0)),
                spec((2, BN, 128), lambda i: (0, i, 0)),
                spec((2, BN, 16), lambda i: (0, i, 0)),
                spec((2, BN, 16), lambda i: (0, i, 0)),
                spec((BN, 16), lambda i: (i, 0)),
                wspec((128, 128)), wspec((128, 128)), wspec((1, 128)),
                wspec((128, 128)), wspec((1, 128)),
                wspec(wA.shape), wspec(wB.shape), wspec(pb1.shape),
            ],
            out_specs=out_specs,
            out_shape=out_shape,
        )(h, parts[0][0], parts[1][0], parts[0][1], parts[1][1], pos16,
          lp["hW1"][:, :128].T, lp["hW1"][:, 128:].T, _row(lp["hb1"]),
          lp["hW2"].T, _row(lp["hb2"]),
          wA, wB, pb1)
        if not last:
            h, pos16, Hd, Hs = outs
        else:
            h, pos16, s_attn = outs

    nlog, y, cls = pl.pallas_call(
        functools.partial(_pool_body, nbN),
        grid=(3 * nbN,),
        in_specs=[
            spec((BN, 128), lambda i: (jnp.maximum(i - 2 * nbN, 0), 0)),
            spec((BN, 1), lambda i: (i % nbN, 0)),
            spec((BN, 1), lambda i: (i % nbN, 0)),
            wspec((B, 16)), wspec((16, 32)), wspec((1, 32)),
            wspec((1, 32)), wspec((1, 32)),
            wspec((B, 1)), wspec((1000, 32)), wspec((64, 128)),
            wspec((128, 128)), wspec((1, 128)),
            wspec((128, 1)), wspec((1, 1)),
            wspec((192, 128)), wspec((1, 128)),
            wspec((128, 1)), wspec((1, 1)),
            wspec((192, 128)), wspec((1, 128)),
            wspec((128, 1)), wspec((1, 1)),
        ],
        out_specs=[spec((BN, 1), lambda i: (jnp.maximum(i - 2 * nbN, 0), 0)),
                   spec((B, 1)), spec((B, 1))],
        out_shape=[jax.ShapeDtypeStruct((N, 1), f32),
                   jax.ShapeDtypeStruct((B, 1), f32),
                   jax.ShapeDtypeStruct((B, 1), f32)],
        scratch_shapes=[pltpu.VMEM((B, 128), f32),
                        pltpu.VMEM((1, 32), f32),
                        pltpu.VMEM((1, 32), f32),
                        pltpu.VMEM((B, 64), f32),
                        pltpu.VMEM((B, 128), f32)],
    )(h, s_attn, batch2d,
      u, p["uW"].T, _row(p["ub"]), _row(p["u_ln_g"]), _row(p["u_ln_b"]),
      pid2d, p["pid_table"], p["nW1"][:, 128:].T,
      p["nW1"][:, :128].T, _row(p["nb1"]), p["nW2"].T, _row(p["nb2"]),
      p["rW1"].T, _row(p["rb1"]), p["rW2"].T, _row(p["rb2"]),
      p["cW1"].T, _row(p["cb1"]), p["cW2"].T, _row(p["cb2"]))

    return (y[:, 0], cls[:, 0], nlog[:, 0], h, pos16[:, :3])
